# Initial kernel scaffold; baseline (speedup 1.0000x reference)
import jax
import jax.numpy as jnp
from jax.experimental import pallas as pl


def kernel(node_features, edge_index, edge_label_index, W1_1, b1_1, W2_1, b2_1, W1_2, b1_2, W2_2, b2_2):
    raise NotImplementedError("write your pallas kernel here")


# speedup vs baseline: 21.7359x; 21.7359x over previous
import functools

import jax
import jax.numpy as jnp
from jax import lax
from jax.experimental import pallas as pl
from jax.experimental.pallas import tpu as pltpu
from jax.experimental.pallas import tpu_sc as plsc

R_CONST = 2.0
T_CONST = 1.0

NCORES = 2
NSUB = 16
NW = NCORES * NSUB


def _node_pad(n):
    per_tile = -(-n // NSUB)
    per_tile = -(-per_tile // 8) * 8
    return per_tile * NSUB


def _make_deg_kernel(NP, NCH, CH):
    mesh = plsc.VectorSubcoreMesh(core_axis_name="c", subcore_axis_name="s")

    @functools.partial(
        pl.kernel,
        out_type=jax.ShapeDtypeStruct((NCORES, NP, 16), jnp.float32),
        mesh=mesh,
        compiler_params=pltpu.CompilerParams(use_tc_tiling_on_sc=False, needs_layout_passes=False),
        scratch_types=[
            pltpu.VMEM((NCH, CH), jnp.int32),
            pltpu.VMEM((CH, 16), jnp.float32),
            pltpu.VMEM_SHARED((NP, 16), jnp.float32),
            pltpu.SemaphoreType.DMA,
        ],
    )
    def deg_kernel(dst3_h, zer16_h, ones_h, out_h, didx, ones_v, acc, sem0):
        cid = lax.axis_index("c")
        sid = lax.axis_index("s")
        wid = cid * NSUB + sid
        rows = NP // NSUB
        lo = sid * rows
        pltpu.sync_copy(zer16_h.at[pl.ds(lo, rows)], acc.at[pl.ds(lo, rows)])
        pltpu.sync_copy(dst3_h.at[wid], didx)
        pltpu.sync_copy(ones_h, ones_v)
        plsc.subcore_barrier()

        @pl.loop(0, NCH, step=8)
        def _wave(k):
            for b in range(8):
                @pl.when(k + b < NCH)
                def _fire():
                    pltpu.async_copy(ones_v, acc.at[didx.at[k + b]], sem0,
                                     add=True)
            for b in range(8):
                @pl.when(k + b < NCH)
                def _drain():
                    pltpu.make_async_copy(ones_v, acc.at[didx.at[k + b]],
                                          sem0).wait()

        plsc.subcore_barrier()
        pltpu.sync_copy(acc.at[pl.ds(lo, rows)],
                        out_h.at[cid, pl.ds(lo, rows)])

    return deg_kernel


def _make_prop_kernel(NP, N, NCH, CH, D):
    mesh = plsc.VectorSubcoreMesh(core_axis_name="c", subcore_axis_name="s")

    @functools.partial(
        pl.kernel,
        out_type=jax.ShapeDtypeStruct((NCORES, NP, D), jnp.float32),
        mesh=mesh,
        compiler_params=pltpu.CompilerParams(use_tc_tiling_on_sc=False, needs_layout_passes=False),
        scratch_types=[
            pltpu.VMEM((NCH, CH), jnp.int32),
            pltpu.VMEM((NCH, CH), jnp.int32),
            pltpu.VMEM((2, CH, D), jnp.float32),
            pltpu.VMEM_SHARED((NP, D), jnp.float32),
            pltpu.SemaphoreType.DMA,
            pltpu.SemaphoreType.DMA,
        ],
    )
    def prop_kernel(src3_h, dst3_h, ystack_h, zer_h, out_h,
                    sidx, didx, rows_v, acc, sem0, sem1):
        cid = lax.axis_index("c")
        sid = lax.axis_index("s")
        rows = NP // NSUB
        lo = sid * rows
        tbl = ystack_h.at[cid]
        pltpu.sync_copy(zer_h.at[pl.ds(lo, rows)], acc.at[pl.ds(lo, rows)])
        pltpu.sync_copy(src3_h.at[sid], sidx)
        pltpu.sync_copy(dst3_h.at[sid], didx)
        plsc.subcore_barrier()

        sems = (sem0, sem1)
        pltpu.async_copy(tbl.at[sidx.at[0]], rows_v.at[0], sem0)

        @pl.loop(0, NCH, step=2)
        def _pipe(k):
            for b in range(2):
                kk = k + b

                @pl.when(kk + 1 < NCH)
                def _fire_next():
                    pltpu.async_copy(tbl.at[sidx.at[kk + 1]],
                                     rows_v.at[1 - b], sems[1 - b])
                pltpu.make_async_copy(tbl.at[sidx.at[kk]],
                                      rows_v.at[b], sems[b]).wait()
                pltpu.sync_copy(rows_v.at[b], acc.at[didx.at[kk]],
                                add=True)

        plsc.subcore_barrier()
        pltpu.sync_copy(acc.at[pl.ds(lo, rows)],
                        out_h.at[cid, pl.ds(lo, rows)])

    return prop_kernel


def _make_decode_kernel(ELP, NDC, DCH, FW):
    mesh = plsc.VectorSubcoreMesh(core_axis_name="c", subcore_axis_name="s")
    NG = FW // 16

    @functools.partial(
        pl.kernel,
        out_type=jax.ShapeDtypeStruct((ELP,), jnp.float32),
        mesh=mesh,
        compiler_params=pltpu.CompilerParams(use_tc_tiling_on_sc=False, needs_layout_passes=False),
        scratch_types=[
            pltpu.VMEM((NDC, DCH), jnp.int32),
            pltpu.VMEM((NDC, DCH), jnp.int32),
            pltpu.VMEM((2, DCH, FW), jnp.float32),
            pltpu.VMEM((2, DCH, FW), jnp.float32),
            pltpu.VMEM((DCH,), jnp.float32),
            pltpu.SemaphoreType.DMA,
            pltpu.SemaphoreType.DMA,
            pltpu.SemaphoreType.DMA,
            pltpu.SemaphoreType.DMA,
        ],
    )
    def decode_kernel(ein3_h, eout3_h, z_h, out_h,
                      iin, iout, arows, brows, out_v, sa0, sa1, sb0, sb1):
        cid = lax.axis_index("c")
        sid = lax.axis_index("s")
        wid = cid * NSUB + sid
        base = wid * NDC * DCH
        pltpu.sync_copy(ein3_h.at[wid], iin)
        pltpu.sync_copy(eout3_h.at[wid], iout)

        sas = (sa0, sa1)
        sbs = (sb0, sb1)
        pltpu.async_copy(z_h.at[iin.at[0]], arows.at[0], sa0)
        pltpu.async_copy(z_h.at[iout.at[0]], brows.at[0], sb0)

        lane = lax.broadcasted_iota(jnp.int32, (16,), 0)

        @pl.loop(0, NDC, step=2)
        def _pipe(k):
            for b in range(2):
                kk = k + b

                @pl.when(kk < NDC)
                def _step():
                    @pl.when(kk + 1 < NDC)
                    def _fire_next():
                        pltpu.async_copy(z_h.at[iin.at[kk + 1]],
                                         arows.at[1 - b], sas[1 - b])
                        pltpu.async_copy(z_h.at[iout.at[kk + 1]],
                                         brows.at[1 - b], sbs[1 - b])
                    pltpu.make_async_copy(z_h.at[iin.at[kk]],
                                          arows.at[b], sas[b]).wait()
                    pltpu.make_async_copy(z_h.at[iout.at[kk]],
                                          brows.at[b], sbs[b]).wait()

                    @pl.loop(0, DCH // 16)
                    def _group(g):
                        sq = jnp.zeros((16,), jnp.float32)
                        for j in range(16):
                            e = g * 16 + j
                            acc = jnp.zeros((16,), jnp.float32)
                            for t in range(NG):
                                av = arows[b, e, pl.ds(t * 16, 16)]
                                bv = brows[b, e, pl.ds(t * 16, 16)]
                                d = av - bv
                                acc = acc + d * d
                            s = jnp.sum(acc)
                            sq = jnp.where(lane == j, s, sq)
                        pv = 1.0 / (jnp.exp((sq - R_CONST) / T_CONST) + 1.0)
                        out_v[pl.ds(g * 16, 16)] = pv

                    pltpu.sync_copy(out_v,
                                    out_h.at[pl.ds(base + kk * DCH, DCH)])

    return decode_kernel


def _tc_stage1(x, degp, w1, w2, N, D, RB):
    nblk = N // RB

    def body(x_ref, dp_ref, w1_ref, w2_ref, o_ref):
        deg = dp_ref[0, :, 0:1] + dp_ref[1, :, 0:1] + 1.0
        dinv = lax.rsqrt(deg)
        xv = x_ref[...]
        o_ref[0] = jnp.dot(xv, w1_ref[...],
                           preferred_element_type=jnp.float32) * dinv
        o_ref[1] = jnp.dot(xv, w2_ref[...],
                           preferred_element_type=jnp.float32) * dinv

    F = x.shape[1]
    return pl.pallas_call(
        body,
        grid=(nblk,),
        in_specs=[
            pl.BlockSpec((RB, F), lambda i: (i, 0)),
            pl.BlockSpec((2, RB, 16), lambda i: (0, i, 0)),
            pl.BlockSpec((F, D), lambda i: (0, 0)),
            pl.BlockSpec((F, D), lambda i: (0, 0)),
        ],
        out_specs=pl.BlockSpec((2, RB, D), lambda i: (0, i, 0)),
        out_shape=jax.ShapeDtypeStruct((2, N, D), jnp.float32),
    )(x, degp, w1, w2)


def _tc_stage2(p, ystack, degp, b1, b2, w1, w2, N, NP, D, RB):
    nblk = N // RB

    def body(p_ref, y_ref, dp_ref, b1_ref, b2_ref, w1_ref, w2_ref, o_ref):
        deg = dp_ref[0, :, 0:1] + dp_ref[1, :, 0:1] + 1.0
        dinv = lax.rsqrt(deg)
        h1 = jnp.maximum((p_ref[0] + y_ref[0]) * dinv + b1_ref[...], 0.0)
        h2 = jnp.maximum((p_ref[1] + y_ref[1]) * dinv + b2_ref[...], 0.0)
        o_ref[0] = jnp.dot(h1, w1_ref[...],
                           preferred_element_type=jnp.float32) * dinv
        o_ref[1] = jnp.dot(h2, w2_ref[...],
                           preferred_element_type=jnp.float32) * dinv

    return pl.pallas_call(
        body,
        grid=(nblk,),
        in_specs=[
            pl.BlockSpec((2, RB, D), lambda i: (0, i, 0)),
            pl.BlockSpec((2, RB, D), lambda i: (0, i, 0)),
            pl.BlockSpec((2, RB, 16), lambda i: (0, i, 0)),
            pl.BlockSpec((1, D), lambda i: (0, 0)),
            pl.BlockSpec((1, D), lambda i: (0, 0)),
            pl.BlockSpec((D, D), lambda i: (0, 0)),
            pl.BlockSpec((D, D), lambda i: (0, 0)),
        ],
        out_specs=pl.BlockSpec((2, RB, D), lambda i: (0, i, 0)),
        out_shape=jax.ShapeDtypeStruct((2, N, D), jnp.float32),
    )(p, ystack, degp, b1, b2, w1, w2)


def _tc_stage3(p, ystack, degp, b1, b2, N, NP, D, RB):
    nblk = N // RB

    def body(p_ref, y_ref, dp_ref, b1_ref, b2_ref, o_ref):
        deg = dp_ref[0, :, 0:1] + dp_ref[1, :, 0:1] + 1.0
        dinv = lax.rsqrt(deg)
        z1 = (p_ref[0] + y_ref[0]) * dinv + b1_ref[...]
        z2 = (p_ref[1] + y_ref[1]) * dinv + b2_ref[...]
        o_ref[...] = jnp.concatenate([z1, z2], axis=1)

    return pl.pallas_call(
        body,
        grid=(nblk,),
        in_specs=[
            pl.BlockSpec((2, RB, D), lambda i: (0, i, 0)),
            pl.BlockSpec((2, RB, D), lambda i: (0, i, 0)),
            pl.BlockSpec((2, RB, 16), lambda i: (0, i, 0)),
            pl.BlockSpec((1, D), lambda i: (0, 0)),
            pl.BlockSpec((1, D), lambda i: (0, 0)),
        ],
        out_specs=pl.BlockSpec((RB, 2 * D), lambda i: (i, 0)),
        out_shape=jax.ShapeDtypeStruct((N, 2 * D), jnp.float32),
    )(p, ystack, degp, b1, b2)


def kernel(node_features, edge_index, edge_label_index,
           W1_1, b1_1, W2_1, b2_1, W1_2, b1_2, W2_2, b2_2):
    x = node_features.astype(jnp.float32)
    N, F = x.shape
    E = edge_index.shape[1]
    EL = edge_label_index.shape[1]
    D = W1_1.shape[1]
    FW = 2 * D

    NP = _node_pad(N)
    RB = 1000
    assert N % RB == 0 and E % NW == 0

    CH = 80
    EPW = E // NW
    NCH_D = EPW // CH
    EPS = E // NSUB
    NCH_P = EPS // CH
    assert EPW % CH == 0 and EPS % CH == 0

    DCH = 128
    NDC = -(-EL // (NW * DCH))
    ELP = NW * NDC * DCH

    src3 = edge_index[0].reshape(NSUB, NCH_P, CH)
    dst3 = edge_index[1].reshape(NSUB, NCH_P, CH)
    dst3d = edge_index[1].reshape(NW, NCH_D, CH)
    eli = jnp.pad(edge_label_index, ((0, 0), (0, ELP - EL)))
    ein3 = eli[0].reshape(NW, NDC, DCH)
    eout3 = eli[1].reshape(NW, NDC, DCH)

    b1_1r = b1_1[None, :]
    b2_1r = b2_1[None, :]
    b1_2r = b1_2[None, :]
    b2_2r = b2_2[None, :]

    zer16 = jnp.zeros((NP, 16), jnp.float32)
    zerD = jnp.zeros((NP, D), jnp.float32)
    ones16 = jnp.ones((CH, 16), jnp.float32)

    deg_k = _make_deg_kernel(NP, NCH_D, CH)
    prop_k = _make_prop_kernel(NP, N, NCH_P, CH, D)
    dec_k = _make_decode_kernel(ELP, NDC, DCH, FW)

    degp = deg_k(dst3d, zer16, ones16)
    y1 = _tc_stage1(x, degp, W1_1, W2_1, N, D, RB)
    p1 = prop_k(src3, dst3, y1, zerD)
    y2 = _tc_stage2(p1, y1, degp, b1_1r, b2_1r, W1_2, W2_2, N, NP, D, RB)
    p2 = prop_k(src3, dst3, y2, zerD)
    z = _tc_stage3(p2, y2, degp, b1_2r, b2_2r, N, NP, D, RB)
    probs = dec_k(ein3, eout3, z)
    return probs[:EL]


# speedup vs baseline: 21.8959x; 1.0074x over previous
import functools

import jax
import jax.numpy as jnp
from jax import lax
from jax.experimental import pallas as pl
from jax.experimental.pallas import tpu as pltpu
from jax.experimental.pallas import tpu_sc as plsc

R_CONST = 2.0
T_CONST = 1.0

NCORES = 2
NSUB = 16
NW = NCORES * NSUB


def _node_pad(n):
    per_tile = -(-n // NSUB)
    per_tile = -(-per_tile // 8) * 8
    return per_tile * NSUB


def _make_deg_kernel(NP, NCH, CH):
    mesh = plsc.VectorSubcoreMesh(core_axis_name="c", subcore_axis_name="s")

    @functools.partial(
        pl.kernel,
        out_type=jax.ShapeDtypeStruct((NCORES, NP, 16), jnp.float32),
        mesh=mesh,
        compiler_params=pltpu.CompilerParams(use_tc_tiling_on_sc=False, needs_layout_passes=False),
        scratch_types=[
            pltpu.VMEM((NCH, CH), jnp.int32),
            pltpu.VMEM((CH, 16), jnp.float32),
            pltpu.VMEM_SHARED((NP, 16), jnp.float32),
            pltpu.SemaphoreType.DMA,
        ],
    )
    def deg_kernel(dst3_h, zer16_h, ones_h, out_h, didx, ones_v, acc, sem0):
        cid = lax.axis_index("c")
        sid = lax.axis_index("s")
        wid = cid * NSUB + sid
        rows = NP // NSUB
        lo = sid * rows
        pltpu.sync_copy(zer16_h.at[pl.ds(lo, rows)], acc.at[pl.ds(lo, rows)])
        pltpu.sync_copy(dst3_h.at[wid], didx)
        pltpu.sync_copy(ones_h, ones_v)
        plsc.subcore_barrier()

        @pl.loop(0, NCH, step=8)
        def _wave(k):
            for b in range(8):
                @pl.when(k + b < NCH)
                def _fire():
                    pltpu.async_copy(ones_v, acc.at[didx.at[k + b]], sem0,
                                     add=True)
            for b in range(8):
                @pl.when(k + b < NCH)
                def _drain():
                    pltpu.make_async_copy(ones_v, acc.at[didx.at[k + b]],
                                          sem0).wait()

        plsc.subcore_barrier()
        pltpu.sync_copy(acc.at[pl.ds(lo, rows)],
                        out_h.at[cid, pl.ds(lo, rows)])

    return deg_kernel


def _make_prop_kernel(NP, N, NCH, CH, D):
    mesh = plsc.VectorSubcoreMesh(core_axis_name="c", subcore_axis_name="s")

    @functools.partial(
        pl.kernel,
        out_type=jax.ShapeDtypeStruct((NCORES, NP, D), jnp.float32),
        mesh=mesh,
        compiler_params=pltpu.CompilerParams(use_tc_tiling_on_sc=False, needs_layout_passes=False),
        scratch_types=[
            pltpu.VMEM((NCH, CH), jnp.int32),
            pltpu.VMEM((NCH, CH), jnp.int32),
            pltpu.VMEM((2, CH, D), jnp.float32),
            pltpu.VMEM_SHARED((NP, D), jnp.float32),
            pltpu.SemaphoreType.DMA,
            pltpu.SemaphoreType.DMA,
            pltpu.SemaphoreType.DMA,
            pltpu.SemaphoreType.DMA,
        ],
    )
    def prop_kernel(src3_h, dst3_h, ystack_h, zer_h, out_h,
                    sidx, didx, rows_v, acc, sg0, sg1, ss0, ss1):
        cid = lax.axis_index("c")
        sid = lax.axis_index("s")
        rows = NP // NSUB
        lo = sid * rows
        tbl = ystack_h.at[cid]
        pltpu.sync_copy(zer_h.at[pl.ds(lo, rows)], acc.at[pl.ds(lo, rows)])
        pltpu.sync_copy(src3_h.at[sid], sidx)
        pltpu.sync_copy(dst3_h.at[sid], didx)
        plsc.subcore_barrier()

        sg = (sg0, sg1)
        ss = (ss0, ss1)
        pltpu.async_copy(tbl.at[sidx.at[0]], rows_v.at[0], sg0)

        @pl.loop(0, NCH, step=2)
        def _pipe(k):
            for b in range(2):
                kk = k + b

                @pl.when(kk >= 1)
                def _drain_prev_scatter():
                    pltpu.make_async_copy(rows_v.at[1 - b],
                                          acc.at[didx.at[kk - 1]],
                                          ss[1 - b]).wait()

                @pl.when(kk + 1 < NCH)
                def _fire_next():
                    pltpu.async_copy(tbl.at[sidx.at[kk + 1]],
                                     rows_v.at[1 - b], sg[1 - b])
                pltpu.make_async_copy(tbl.at[sidx.at[kk]],
                                      rows_v.at[b], sg[b]).wait()
                pltpu.async_copy(rows_v.at[b], acc.at[didx.at[kk]],
                                 ss[b], add=True)

        pltpu.make_async_copy(rows_v.at[(NCH - 1) % 2],
                              acc.at[didx.at[NCH - 1]],
                              ss[(NCH - 1) % 2]).wait()

        plsc.subcore_barrier()
        pltpu.sync_copy(acc.at[pl.ds(lo, rows)],
                        out_h.at[cid, pl.ds(lo, rows)])

    return prop_kernel


def _make_decode_kernel(ELP, NDC, DCH, FW):
    mesh = plsc.VectorSubcoreMesh(core_axis_name="c", subcore_axis_name="s")
    NG = FW // 16

    @functools.partial(
        pl.kernel,
        out_type=jax.ShapeDtypeStruct((ELP,), jnp.float32),
        mesh=mesh,
        compiler_params=pltpu.CompilerParams(use_tc_tiling_on_sc=False, needs_layout_passes=False),
        scratch_types=[
            pltpu.VMEM((NDC, DCH), jnp.int32),
            pltpu.VMEM((NDC, DCH), jnp.int32),
            pltpu.VMEM((3, DCH, FW), jnp.float32),
            pltpu.VMEM((3, DCH, FW), jnp.float32),
            pltpu.VMEM((3, DCH), jnp.float32),
            [pltpu.SemaphoreType.DMA] * 3,
            [pltpu.SemaphoreType.DMA] * 3,
            [pltpu.SemaphoreType.DMA] * 3,
        ],
    )
    def decode_kernel(ein3_h, eout3_h, z_h, out_h,
                      iin, iout, arows, brows, out_v, sas, sbs, sos):
        cid = lax.axis_index("c")
        sid = lax.axis_index("s")
        wid = cid * NSUB + sid
        base = wid * NDC * DCH
        pltpu.sync_copy(ein3_h.at[wid], iin)
        pltpu.sync_copy(eout3_h.at[wid], iout)

        for i in range(3):
            pltpu.async_copy(z_h.at[iin.at[i]], arows.at[i], sas[i])
            pltpu.async_copy(z_h.at[iout.at[i]], brows.at[i], sbs[i])

        lane = lax.broadcasted_iota(jnp.int32, (16,), 0)

        @pl.loop(0, NDC, step=3)
        def _pipe(k):
            for b in range(3):
                kk = k + b

                @pl.when(kk < NDC)
                def _step():
                    pltpu.make_async_copy(z_h.at[iin.at[kk]],
                                          arows.at[b], sas[b]).wait()
                    pltpu.make_async_copy(z_h.at[iout.at[kk]],
                                          brows.at[b], sbs[b]).wait()

                    @pl.when(kk >= 3)
                    def _drain_out():
                        pltpu.make_async_copy(
                            out_v.at[b],
                            out_h.at[pl.ds(base + (kk - 3) * DCH, DCH)],
                            sos[b]).wait()

                    @pl.loop(0, DCH // 16)
                    def _group(g):
                        sq = jnp.zeros((16,), jnp.float32)
                        for j in range(16):
                            e = g * 16 + j
                            acc = jnp.zeros((16,), jnp.float32)
                            for t in range(NG):
                                av = arows[b, e, pl.ds(t * 16, 16)]
                                bv = brows[b, e, pl.ds(t * 16, 16)]
                                d = av - bv
                                acc = acc + d * d
                            s = jnp.sum(acc)
                            sq = jnp.where(lane == j, s, sq)
                        pv = 1.0 / (jnp.exp((sq - R_CONST) / T_CONST) + 1.0)
                        out_v[b, pl.ds(g * 16, 16)] = pv

                    pltpu.async_copy(out_v.at[b],
                                     out_h.at[pl.ds(base + kk * DCH, DCH)],
                                     sos[b])

                    @pl.when(kk + 3 < NDC)
                    def _fire_next():
                        pltpu.async_copy(z_h.at[iin.at[kk + 3]],
                                         arows.at[b], sas[b])
                        pltpu.async_copy(z_h.at[iout.at[kk + 3]],
                                         brows.at[b], sbs[b])

        for kk in range(max(NDC - 3, 0), NDC):
            bb = kk % 3
            pltpu.make_async_copy(out_v.at[bb],
                                  out_h.at[pl.ds(base + kk * DCH, DCH)],
                                  sos[bb]).wait()

    return decode_kernel


def _tc_stage1(x, degp, w1, w2, N, D, RB):
    nblk = N // RB

    def body(x_ref, dp_ref, w1_ref, w2_ref, o_ref):
        deg = dp_ref[0, :, 0:1] + dp_ref[1, :, 0:1] + 1.0
        dinv = lax.rsqrt(deg)
        xv = x_ref[...]
        o_ref[0] = jnp.dot(xv, w1_ref[...],
                           preferred_element_type=jnp.float32) * dinv
        o_ref[1] = jnp.dot(xv, w2_ref[...],
                           preferred_element_type=jnp.float32) * dinv

    F = x.shape[1]
    return pl.pallas_call(
        body,
        grid=(nblk,),
        in_specs=[
            pl.BlockSpec((RB, F), lambda i: (i, 0)),
            pl.BlockSpec((2, RB, 16), lambda i: (0, i, 0)),
            pl.BlockSpec((F, D), lambda i: (0, 0)),
            pl.BlockSpec((F, D), lambda i: (0, 0)),
        ],
        out_specs=pl.BlockSpec((2, RB, D), lambda i: (0, i, 0)),
        out_shape=jax.ShapeDtypeStruct((2, N, D), jnp.float32),
    )(x, degp, w1, w2)


def _tc_stage2(p, ystack, degp, b1, b2, w1, w2, N, NP, D, RB):
    nblk = N // RB

    def body(p_ref, y_ref, dp_ref, b1_ref, b2_ref, w1_ref, w2_ref, o_ref):
        deg = dp_ref[0, :, 0:1] + dp_ref[1, :, 0:1] + 1.0
        dinv = lax.rsqrt(deg)
        h1 = jnp.maximum((p_ref[0] + y_ref[0]) * dinv + b1_ref[...], 0.0)
        h2 = jnp.maximum((p_ref[1] + y_ref[1]) * dinv + b2_ref[...], 0.0)
        o_ref[0] = jnp.dot(h1, w1_ref[...],
                           preferred_element_type=jnp.float32) * dinv
        o_ref[1] = jnp.dot(h2, w2_ref[...],
                           preferred_element_type=jnp.float32) * dinv

    return pl.pallas_call(
        body,
        grid=(nblk,),
        in_specs=[
            pl.BlockSpec((2, RB, D), lambda i: (0, i, 0)),
            pl.BlockSpec((2, RB, D), lambda i: (0, i, 0)),
            pl.BlockSpec((2, RB, 16), lambda i: (0, i, 0)),
            pl.BlockSpec((1, D), lambda i: (0, 0)),
            pl.BlockSpec((1, D), lambda i: (0, 0)),
            pl.BlockSpec((D, D), lambda i: (0, 0)),
            pl.BlockSpec((D, D), lambda i: (0, 0)),
        ],
        out_specs=pl.BlockSpec((2, RB, D), lambda i: (0, i, 0)),
        out_shape=jax.ShapeDtypeStruct((2, N, D), jnp.float32),
    )(p, ystack, degp, b1, b2, w1, w2)


def _tc_stage3(p, ystack, degp, b1, b2, N, NP, D, RB):
    nblk = N // RB

    def body(p_ref, y_ref, dp_ref, b1_ref, b2_ref, o_ref):
        deg = dp_ref[0, :, 0:1] + dp_ref[1, :, 0:1] + 1.0
        dinv = lax.rsqrt(deg)
        z1 = (p_ref[0] + y_ref[0]) * dinv + b1_ref[...]
        z2 = (p_ref[1] + y_ref[1]) * dinv + b2_ref[...]
        o_ref[...] = jnp.concatenate([z1, z2], axis=1)

    return pl.pallas_call(
        body,
        grid=(nblk,),
        in_specs=[
            pl.BlockSpec((2, RB, D), lambda i: (0, i, 0)),
            pl.BlockSpec((2, RB, D), lambda i: (0, i, 0)),
            pl.BlockSpec((2, RB, 16), lambda i: (0, i, 0)),
            pl.BlockSpec((1, D), lambda i: (0, 0)),
            pl.BlockSpec((1, D), lambda i: (0, 0)),
        ],
        out_specs=pl.BlockSpec((RB, 2 * D), lambda i: (i, 0)),
        out_shape=jax.ShapeDtypeStruct((N, 2 * D), jnp.float32),
    )(p, ystack, degp, b1, b2)


def kernel(node_features, edge_index, edge_label_index,
           W1_1, b1_1, W2_1, b2_1, W1_2, b1_2, W2_2, b2_2):
    x = node_features.astype(jnp.float32)
    N, F = x.shape
    E = edge_index.shape[1]
    EL = edge_label_index.shape[1]
    D = W1_1.shape[1]
    FW = 2 * D

    NP = _node_pad(N)
    RB = 1000
    assert N % RB == 0 and E % NW == 0

    CH = 80
    EPW = E // NW
    NCH_D = EPW // CH
    EPS = E // NSUB
    NCH_P = EPS // CH
    assert EPW % CH == 0 and EPS % CH == 0

    DCH = 128
    NDC = -(-EL // (NW * DCH))
    ELP = NW * NDC * DCH

    src3 = edge_index[0].reshape(NSUB, NCH_P, CH)
    dst3 = edge_index[1].reshape(NSUB, NCH_P, CH)
    dst3d = edge_index[1].reshape(NW, NCH_D, CH)
    eli = jnp.pad(edge_label_index, ((0, 0), (0, ELP - EL)))
    ein3 = eli[0].reshape(NW, NDC, DCH)
    eout3 = eli[1].reshape(NW, NDC, DCH)

    b1_1r = b1_1[None, :]
    b2_1r = b2_1[None, :]
    b1_2r = b1_2[None, :]
    b2_2r = b2_2[None, :]

    zer16 = jnp.zeros((NP, 16), jnp.float32)
    zerD = jnp.zeros((NP, D), jnp.float32)
    ones16 = jnp.ones((CH, 16), jnp.float32)

    deg_k = _make_deg_kernel(NP, NCH_D, CH)
    prop_k = _make_prop_kernel(NP, N, NCH_P, CH, D)
    dec_k = _make_decode_kernel(ELP, NDC, DCH, FW)

    degp = deg_k(dst3d, zer16, ones16)
    y1 = _tc_stage1(x, degp, W1_1, W2_1, N, D, RB)
    p1 = prop_k(src3, dst3, y1, zerD)
    y2 = _tc_stage2(p1, y1, degp, b1_1r, b2_1r, W1_2, W2_2, N, NP, D, RB)
    p2 = prop_k(src3, dst3, y2, zerD)
    z = _tc_stage3(p2, y2, degp, b1_2r, b2_2r, N, NP, D, RB)
    probs = dec_k(ein3, eout3, z)
    return probs[:EL]
